# Initial kernel scaffold; baseline (speedup 1.0000x reference)
import jax
import jax.numpy as jnp
from jax.experimental import pallas as pl


def kernel(raw_features, wl_role_ids, init_pos_ids, hop_dis_ids, W_raw, b_raw, wl_table, pos_table, hop_table, gamma, beta):
    raise NotImplementedError("write your pallas kernel here")


# speedup vs baseline: 3.5130x; 3.5130x over previous
import functools

import jax
import jax.numpy as jnp
from jax import lax
from jax.experimental import pallas as pl
from jax.experimental.pallas import tpu as pltpu
from jax.experimental.pallas import tpu_sc as plsc

NUM_FEATURES = 128
HIDDEN = 128
EPS = 1e-12

N_ROWS = 4096 * 50
NUM_CORES = 2
NUM_SUBCORES = 16
NW = NUM_CORES * NUM_SUBCORES
ROWS_PER_W = N_ROWS // NW
CHUNK = 128
N_CHUNKS = ROWS_PER_W // CHUNK
LANES = 16

TC_BLOCK = 2048


def _sc_gather_sum(wl_table, pos_table, hop_table, wl_ids, pos_ids, hop_ids):
    mesh = plsc.VectorSubcoreMesh(core_axis_name="c", subcore_axis_name="s")

    @functools.partial(
        pl.kernel,
        mesh=mesh,
        out_type=jax.ShapeDtypeStruct((N_ROWS, HIDDEN), jnp.float32),
        scratch_types=[
            pltpu.VMEM((CHUNK,), jnp.int32),
            pltpu.VMEM((CHUNK,), jnp.int32),
            pltpu.VMEM((CHUNK,), jnp.int32),
            pltpu.VMEM((CHUNK, HIDDEN), jnp.float32),
            pltpu.VMEM((CHUNK, HIDDEN), jnp.float32),
            pltpu.VMEM((CHUNK, HIDDEN), jnp.float32),
            pltpu.SemaphoreType.DMA,
            pltpu.SemaphoreType.DMA,
            pltpu.SemaphoreType.DMA,
        ],
    )
    def gather_kernel(wl_hbm, pos_hbm, hop_hbm, wl_ids_hbm, pos_ids_hbm,
                      hop_ids_hbm, out_hbm, wl_idx, pos_idx, hop_idx,
                      acc, buf_b, buf_c, sem_a, sem_b, sem_c):
        wid = lax.axis_index("s") * NUM_CORES + lax.axis_index("c")
        base = wid * ROWS_PER_W

        def chunk_body(j, carry):
            row0 = base + j * CHUNK
            pltpu.sync_copy(wl_ids_hbm.at[pl.ds(row0, CHUNK)], wl_idx)
            pltpu.sync_copy(pos_ids_hbm.at[pl.ds(row0, CHUNK)], pos_idx)
            pltpu.sync_copy(hop_ids_hbm.at[pl.ds(row0, CHUNK)], hop_idx)
            ca = pltpu.async_copy(wl_hbm.at[wl_idx], acc, sem_a)
            cb = pltpu.async_copy(pos_hbm.at[pos_idx], buf_b, sem_b)
            cc = pltpu.async_copy(hop_hbm.at[hop_idx], buf_c, sem_c)
            ca.wait()
            cb.wait()
            cc.wait()

            def add_row(r, inner):
                for cv in range(HIDDEN // LANES):
                    sl = pl.ds(cv * LANES, LANES)
                    acc[r, sl] = acc[r, sl] + (buf_b[r, sl] + buf_c[r, sl])
                return inner

            lax.fori_loop(0, CHUNK, add_row, 0, unroll=False)
            pltpu.sync_copy(acc, out_hbm.at[pl.ds(row0, CHUNK)])
            return carry

        lax.fori_loop(0, N_CHUNKS, chunk_body, 0, unroll=False)

    return gather_kernel(wl_table, pos_table, hop_table, wl_ids, pos_ids, hop_ids)


def _tc_finish(raw_flat, W, b, sum3, gamma, beta):
    grid = (N_ROWS // TC_BLOCK,)

    def body(raw_ref, w_ref, b_ref, s_ref, g_ref, bt_ref, o_ref):
        x = jnp.dot(raw_ref[...], w_ref[...], preferred_element_type=jnp.float32)
        x = x + b_ref[...] + s_ref[...]
        mean = jnp.mean(x, axis=-1, keepdims=True)
        xc = x - mean
        var = jnp.mean(xc * xc, axis=-1, keepdims=True)
        inv = lax.rsqrt(var + EPS)
        o_ref[...] = xc * inv * g_ref[...] + bt_ref[...]

    return pl.pallas_call(
        body,
        grid=grid,
        in_specs=[
            pl.BlockSpec((TC_BLOCK, NUM_FEATURES), lambda i: (i, 0)),
            pl.BlockSpec((NUM_FEATURES, HIDDEN), lambda i: (0, 0)),
            pl.BlockSpec((1, HIDDEN), lambda i: (0, 0)),
            pl.BlockSpec((TC_BLOCK, HIDDEN), lambda i: (i, 0)),
            pl.BlockSpec((1, HIDDEN), lambda i: (0, 0)),
            pl.BlockSpec((1, HIDDEN), lambda i: (0, 0)),
        ],
        out_specs=pl.BlockSpec((TC_BLOCK, HIDDEN), lambda i: (i, 0)),
        out_shape=jax.ShapeDtypeStruct((N_ROWS, HIDDEN), jnp.float32),
    )(raw_flat, W, b, sum3, gamma, beta)


def kernel(raw_features, wl_role_ids, init_pos_ids, hop_dis_ids, W_raw, b_raw,
           wl_table, pos_table, hop_table, gamma, beta):
    B, L, F = raw_features.shape
    raw_flat = raw_features.reshape(B * L, F)
    wl_ids = wl_role_ids.reshape(-1).astype(jnp.int32)
    pos_ids = init_pos_ids.reshape(-1).astype(jnp.int32)
    hop_ids = hop_dis_ids.reshape(-1).astype(jnp.int32)
    sum3 = _sc_gather_sum(wl_table, pos_table, hop_table, wl_ids, pos_ids, hop_ids)
    out = _tc_finish(raw_flat, W_raw, b_raw.reshape(1, HIDDEN), sum3,
                     gamma.reshape(1, HIDDEN), beta.reshape(1, HIDDEN))
    return out.reshape(B, L, HIDDEN)


# speedup vs baseline: 4.2595x; 1.2125x over previous
import functools

import jax
import jax.numpy as jnp
from jax import lax
from jax.experimental import pallas as pl
from jax.experimental.pallas import tpu as pltpu
from jax.experimental.pallas import tpu_sc as plsc

NUM_FEATURES = 128
HIDDEN = 128
EPS = 1e-12

N_ROWS = 4096 * 50
NUM_CORES = 2
NUM_SUBCORES = 16
NW = NUM_CORES * NUM_SUBCORES
ROWS_PER_W = N_ROWS // NW
CHUNK = 128
N_CHUNKS = ROWS_PER_W // CHUNK
LANES = 16

TC_BLOCK = 2048


def _sc_gather_sum(wl_table, pos_table, hop_table, wl_ids, pos_ids, hop_ids):
    mesh = plsc.VectorSubcoreMesh(core_axis_name="c", subcore_axis_name="s")

    @functools.partial(
        pl.kernel,
        mesh=mesh,
        out_type=jax.ShapeDtypeStruct((N_ROWS, HIDDEN), jnp.float32),
        scratch_types=[
            pltpu.VMEM((CHUNK,), jnp.int32),
            pltpu.VMEM((CHUNK,), jnp.int32),
            pltpu.VMEM((CHUNK,), jnp.int32),
            pltpu.VMEM((CHUNK, HIDDEN), jnp.float32),
            pltpu.VMEM((CHUNK, HIDDEN), jnp.float32),
            pltpu.VMEM((CHUNK, HIDDEN), jnp.float32),
            pltpu.SemaphoreType.DMA,
            pltpu.SemaphoreType.DMA,
            pltpu.SemaphoreType.DMA,
        ],
    )
    def gather_kernel(wl_hbm, pos_hbm, hop_hbm, wl_ids_hbm, pos_ids_hbm,
                      hop_ids_hbm, out_hbm, wl_idx, pos_idx, hop_idx,
                      acc, buf_b, buf_c, sem_a, sem_b, sem_c):
        wid = lax.axis_index("s") * NUM_CORES + lax.axis_index("c")
        base = wid * ROWS_PER_W

        def chunk_body(j, carry):
            row0 = base + j * CHUNK
            pltpu.sync_copy(wl_ids_hbm.at[pl.ds(row0, CHUNK)], wl_idx)
            pltpu.sync_copy(pos_ids_hbm.at[pl.ds(row0, CHUNK)], pos_idx)
            pltpu.sync_copy(hop_ids_hbm.at[pl.ds(row0, CHUNK)], hop_idx)
            ca = pltpu.async_copy(wl_hbm.at[wl_idx], acc, sem_a)
            cb = pltpu.async_copy(pos_hbm.at[pos_idx], buf_b, sem_b)
            cc = pltpu.async_copy(hop_hbm.at[hop_idx], buf_c, sem_c)
            ca.wait()
            cb.wait()
            cc.wait()

            def add_row(r, inner):
                for cv in range(HIDDEN // LANES):
                    sl = pl.ds(cv * LANES, LANES)
                    acc[r, sl] = acc[r, sl] + (buf_b[r, sl] + buf_c[r, sl])
                return inner

            lax.fori_loop(0, CHUNK, add_row, 0, unroll=False)
            pltpu.sync_copy(acc, out_hbm.at[pl.ds(row0, CHUNK)])
            return carry

        lax.fori_loop(0, N_CHUNKS, chunk_body, 0, unroll=False)

    return gather_kernel(wl_table, pos_table, hop_table, wl_ids, pos_ids, hop_ids)


SEQ = 50
TC_BB = 32


def _tc_finish(raw, W, b, sum3, gamma, beta):
    grid = (4096 // TC_BB,)
    rows = TC_BB * SEQ

    def body(raw_ref, w_ref, b_ref, s_ref, g_ref, bt_ref, o_ref):
        raw2 = raw_ref[...].reshape(rows, NUM_FEATURES)
        x = jnp.dot(raw2, w_ref[...], preferred_element_type=jnp.float32)
        x = x + b_ref[...] + s_ref[...]
        mean = jnp.mean(x, axis=-1, keepdims=True)
        xc = x - mean
        var = jnp.mean(xc * xc, axis=-1, keepdims=True)
        inv = lax.rsqrt(var + EPS)
        res = xc * inv * g_ref[...] + bt_ref[...]
        o_ref[...] = res.reshape(TC_BB, SEQ, HIDDEN)

    return pl.pallas_call(
        body,
        grid=grid,
        in_specs=[
            pl.BlockSpec((TC_BB, SEQ, NUM_FEATURES), lambda i: (i, 0, 0)),
            pl.BlockSpec((NUM_FEATURES, HIDDEN), lambda i: (0, 0)),
            pl.BlockSpec((1, HIDDEN), lambda i: (0, 0)),
            pl.BlockSpec((rows, HIDDEN), lambda i: (i, 0)),
            pl.BlockSpec((1, HIDDEN), lambda i: (0, 0)),
            pl.BlockSpec((1, HIDDEN), lambda i: (0, 0)),
        ],
        out_specs=pl.BlockSpec((TC_BB, SEQ, HIDDEN), lambda i: (i, 0, 0)),
        out_shape=jax.ShapeDtypeStruct((4096, SEQ, HIDDEN), jnp.float32),
    )(raw, W, b, sum3, gamma, beta)


def kernel(raw_features, wl_role_ids, init_pos_ids, hop_dis_ids, W_raw, b_raw,
           wl_table, pos_table, hop_table, gamma, beta):
    wl_ids = wl_role_ids.reshape(-1).astype(jnp.int32)
    pos_ids = init_pos_ids.reshape(-1).astype(jnp.int32)
    hop_ids = hop_dis_ids.reshape(-1).astype(jnp.int32)
    sum3 = _sc_gather_sum(wl_table, pos_table, hop_table, wl_ids, pos_ids, hop_ids)
    return _tc_finish(raw_features, W_raw, b_raw.reshape(1, HIDDEN), sum3,
                      gamma.reshape(1, HIDDEN), beta.reshape(1, HIDDEN))


# speedup vs baseline: 5.3156x; 1.2479x over previous
import functools

import jax
import jax.numpy as jnp
from jax import lax
from jax.experimental import pallas as pl
from jax.experimental.pallas import tpu as pltpu
from jax.experimental.pallas import tpu_sc as plsc

NUM_FEATURES = 128
HIDDEN = 128
EPS = 1e-12

N_ROWS = 4096 * 50
NUM_CORES = 2
NUM_SUBCORES = 16
NW = NUM_CORES * NUM_SUBCORES
ROWS_PER_W = N_ROWS // NW
CHUNK = 64
N_CHUNKS = ROWS_PER_W // CHUNK
NBUF = 3
LANES = 16


def _sc_gather_sum(wl_table, pos_table, hop_table, wl_ids, pos_ids, hop_ids):
    mesh = plsc.VectorSubcoreMesh(core_axis_name="c", subcore_axis_name="s")

    buf_t = pltpu.VMEM((CHUNK, HIDDEN), jnp.float32)
    idx_t = pltpu.VMEM((ROWS_PER_W,), jnp.int32)
    sem_t = pltpu.SemaphoreType.DMA

    @functools.partial(
        pl.kernel,
        mesh=mesh,
        out_type=jax.ShapeDtypeStruct((N_ROWS, HIDDEN), jnp.float32),
        scratch_types=(
            [idx_t] * 3
            + [buf_t] * (3 * NBUF)
            + [buf_t] * NBUF
            + [sem_t] * (3 * NBUF)
            + [sem_t] * NBUF
        ),
    )
    def gather_kernel(wl_hbm, pos_hbm, hop_hbm, wl_ids_hbm, pos_ids_hbm,
                      hop_ids_hbm, out_hbm, *scratch):
        idxs = scratch[0:3]
        g_bufs = [scratch[3 + 3 * b: 3 + 3 * b + 3] for b in range(NBUF)]
        o_bufs = scratch[3 + 3 * NBUF: 3 + 4 * NBUF]
        g_sems = [scratch[3 + 4 * NBUF + 3 * b: 3 + 4 * NBUF + 3 * b + 3]
                  for b in range(NBUF)]
        w_sems = scratch[3 + 7 * NBUF: 3 + 8 * NBUF]

        wid = lax.axis_index("s") * NUM_CORES + lax.axis_index("c")
        base = wid * ROWS_PER_W

        pltpu.sync_copy(wl_ids_hbm.at[pl.ds(base, ROWS_PER_W)], idxs[0])
        pltpu.sync_copy(pos_ids_hbm.at[pl.ds(base, ROWS_PER_W)], idxs[1])
        pltpu.sync_copy(hop_ids_hbm.at[pl.ds(base, ROWS_PER_W)], idxs[2])

        tables = (wl_hbm, pos_hbm, hop_hbm)

        def start_gathers(j, b):
            for t in range(3):
                pltpu.async_copy(
                    tables[t].at[idxs[t].at[pl.ds(j * CHUNK, CHUNK)]],
                    g_bufs[b][t], g_sems[b][t])

        def wait_gathers(b):
            for t in range(3):
                pltpu.make_async_copy(
                    out_hbm.at[pl.ds(0, CHUNK)], g_bufs[b][t],
                    g_sems[b][t]).wait()

        def wait_wb(b):
            pltpu.make_async_copy(
                o_bufs[b], out_hbm.at[pl.ds(0, CHUNK)], w_sems[b]).wait()

        def add_chunk(b):
            def add_row(r, inner):
                for cv in range(HIDDEN // LANES):
                    sl = pl.ds(cv * LANES, LANES)
                    o_bufs[b][r, sl] = (g_bufs[b][0][r, sl]
                                        + g_bufs[b][1][r, sl]
                                        + g_bufs[b][2][r, sl])
                return inner

            lax.fori_loop(0, CHUNK, add_row, 0, unroll=False)

        for b in range(NBUF):
            start_gathers(b, b)

        def chunk_body(j, b, wb_wait_traced):
            wait_gathers(b)
            if wb_wait_traced:
                @pl.when(j >= NBUF)
                def _():
                    wait_wb(b)
            add_chunk(b)
            pltpu.async_copy(o_bufs[b],
                             out_hbm.at[pl.ds(base + j * CHUNK, CHUNK)],
                             w_sems[b])

            @pl.when(j + NBUF < N_CHUNKS)
            def _():
                start_gathers(j + NBUF, b)

        def outer_body(g, carry):
            for b in range(NBUF):
                chunk_body(g * NBUF + b, b, True)
            return carry

        lax.fori_loop(0, N_CHUNKS // NBUF, outer_body, 0, unroll=False)
        for j in range(NBUF * (N_CHUNKS // NBUF), N_CHUNKS):
            b = j % NBUF
            wait_gathers(b)
            wait_wb(b)
            add_chunk(b)
            pltpu.async_copy(o_bufs[b],
                             out_hbm.at[pl.ds(base + j * CHUNK, CHUNK)],
                             w_sems[b])
        for b in range(NBUF):
            wait_wb(b)

    return gather_kernel(wl_table, pos_table, hop_table, wl_ids, pos_ids, hop_ids)


SEQ = 50
TC_BB = 32


def _tc_finish(raw, W, b, sum3, gamma, beta):
    grid = (4096 // TC_BB,)
    rows = TC_BB * SEQ

    def body(raw_ref, w_ref, b_ref, s_ref, g_ref, bt_ref, o_ref):
        raw2 = raw_ref[...].reshape(rows, NUM_FEATURES)
        x = jnp.dot(raw2, w_ref[...], preferred_element_type=jnp.float32)
        x = x + b_ref[...] + s_ref[...]
        mean = jnp.mean(x, axis=-1, keepdims=True)
        xc = x - mean
        var = jnp.mean(xc * xc, axis=-1, keepdims=True)
        inv = lax.rsqrt(var + EPS)
        res = xc * inv * g_ref[...] + bt_ref[...]
        o_ref[...] = res.reshape(TC_BB, SEQ, HIDDEN)

    return pl.pallas_call(
        body,
        grid=grid,
        in_specs=[
            pl.BlockSpec((TC_BB, SEQ, NUM_FEATURES), lambda i: (i, 0, 0)),
            pl.BlockSpec((NUM_FEATURES, HIDDEN), lambda i: (0, 0)),
            pl.BlockSpec((1, HIDDEN), lambda i: (0, 0)),
            pl.BlockSpec((rows, HIDDEN), lambda i: (i, 0)),
            pl.BlockSpec((1, HIDDEN), lambda i: (0, 0)),
            pl.BlockSpec((1, HIDDEN), lambda i: (0, 0)),
        ],
        out_specs=pl.BlockSpec((TC_BB, SEQ, HIDDEN), lambda i: (i, 0, 0)),
        out_shape=jax.ShapeDtypeStruct((4096, SEQ, HIDDEN), jnp.float32),
    )(raw, W, b, sum3, gamma, beta)


def kernel(raw_features, wl_role_ids, init_pos_ids, hop_dis_ids, W_raw, b_raw,
           wl_table, pos_table, hop_table, gamma, beta):
    wl_ids = wl_role_ids.astype(jnp.int32).reshape(-1)
    pos_ids = init_pos_ids.astype(jnp.int32).reshape(-1)
    hop_ids = hop_dis_ids.astype(jnp.int32).reshape(-1)
    sum3 = _sc_gather_sum(wl_table, pos_table, hop_table, wl_ids, pos_ids, hop_ids)
    return _tc_finish(raw_features, W_raw, b_raw.reshape(1, HIDDEN), sum3,
                      gamma.reshape(1, HIDDEN), beta.reshape(1, HIDDEN))
